# Initial kernel scaffold; baseline (speedup 1.0000x reference)
import jax
import jax.numpy as jnp
from jax.experimental import pallas as pl


def kernel(V_predict, L_last, V_w, V_mass_no_inf, C_shape, C_init_shape, V_compliance):
    raise NotImplementedError("write your pallas kernel here")


# speedup vs baseline: 269.5283x; 269.5283x over previous
import functools

import jax
import jax.numpy as jnp
from jax import lax
from jax.experimental import pallas as pl
from jax.experimental.pallas import tpu as pltpu
from jax.experimental.pallas import tpu_sc as plsc

NUM_V = 50000
NUM_C = 20000
P = 32

L = 16
NS = 16
GC = 16
SLOTS = GC * P
NG = NUM_C // GC
GPW = -(-NG // NS)
TW = 16
AW = 8
NR = NUM_V // NS


def _sc_body(table_h, idx_h, init_h, vp4_h, out_h,
             acc_s, idx_v, rows_v, init_v, delta_v, sem):
    wid = lax.axis_index("s")
    lids = lax.iota(jnp.int32, L)
    lid32 = lids * P
    lid96 = lids * (P * 3)
    cols = [jnp.full((L,), c, jnp.int32) for c in range(TW)]
    zero16 = jnp.zeros((L,), jnp.float32)

    r0 = wid * NR
    pltpu.sync_copy(vp4_h.at[pl.ds(r0, NR)], acc_s.at[pl.ds(r0, NR)])

    for q in range(SLOTS // L):
        sv = lids + q * L
        jv = lax.shift_right_logical(sv, jnp.int32(7))
        rv = lax.bitwise_and(sv, jnp.int32(127))
        for c in range(3, AW):
            plsc.store_scatter(delta_v, [jv, rv, cols[c]], zero16)

    plsc.subcore_barrier()

    def group_body(g, carry):
        gi = g * jnp.int32(NS) + wid

        @pl.when(gi < NG)
        def _():
            pltpu.sync_copy(idx_h.at[gi], idx_v)
            descs = [pltpu.async_copy(table_h.at[idx_v.at[jnp.int32(j)]],
                                      rows_v.at[jnp.int32(j)], sem)
                     for j in range(SLOTS // 128)]
            pltpu.sync_copy(init_h.at[gi], init_v)
            for d in descs:
                d.wait()

            msum = zero16
            wx = zero16
            wy = zero16
            wz = zero16
            for p in range(P):
                sv = lid32 + p
                jv = lax.shift_right_logical(sv, jnp.int32(7))
                rv = lax.bitwise_and(sv, jnp.int32(127))
                x = plsc.load_gather(rows_v, [jv, rv, cols[0]])
                y = plsc.load_gather(rows_v, [jv, rv, cols[1]])
                z = plsc.load_gather(rows_v, [jv, rv, cols[2]])
                m = plsc.load_gather(rows_v, [jv, rv, cols[3]])
                msum = msum + m
                wx = wx + m * x
                wy = wy + m * y
                wz = wz + m * z
            cx = wx / msum
            cy = wy / msum
            cz = wz / msum

            for p in range(P):
                sv = lid32 + p
                jv = lax.shift_right_logical(sv, jnp.int32(7))
                rv = lax.bitwise_and(sv, jnp.int32(127))
                x = plsc.load_gather(rows_v, [jv, rv, cols[0]])
                y = plsc.load_gather(rows_v, [jv, rv, cols[1]])
                z = plsc.load_gather(rows_v, [jv, rv, cols[2]])
                w = plsc.load_gather(rows_v, [jv, rv, cols[4]])
                cm = plsc.load_gather(rows_v, [jv, rv, cols[5]])
                iv0 = lid96 + 3 * p
                ix = plsc.load_gather(init_v, [iv0])
                iy = plsc.load_gather(init_v, [iv0 + 1])
                iz = plsc.load_gather(init_v, [iv0 + 2])
                s = w / cm
                plsc.store_scatter(delta_v, [jv, rv, cols[0]], s * (ix - x + cx))
                plsc.store_scatter(delta_v, [jv, rv, cols[1]], s * (iy - y + cy))
                plsc.store_scatter(delta_v, [jv, rv, cols[2]], s * (iz - z + cz))

            for j in range(SLOTS // 128):
                pltpu.sync_copy(delta_v.at[jnp.int32(j)],
                                acc_s.at[idx_v.at[jnp.int32(j)]], add=True)

        return carry

    lax.fori_loop(jnp.int32(0), jnp.int32(GPW), group_body, jnp.int32(0))
    plsc.subcore_barrier()
    pltpu.sync_copy(acc_s.at[pl.ds(r0, NR)], out_h.at[pl.ds(r0, NR)])


@jax.jit
def _sc_call(table, idx, initf, vp4):
    mesh = plsc.VectorSubcoreMesh(core_axis_name="c", subcore_axis_name="s",
                                  num_cores=1)
    return pl.kernel(
        _sc_body,
        out_type=jax.ShapeDtypeStruct((NUM_V, AW), jnp.float32),
        mesh=mesh,
        compiler_params=pltpu.CompilerParams(use_tc_tiling_on_sc=False,
                                             needs_layout_passes=False),
        scratch_types=[
            pltpu.VMEM_SHARED((NUM_V, AW), jnp.float32),
            pltpu.VMEM((SLOTS // 128, 128), jnp.int32),
            pltpu.VMEM((SLOTS // 128, 128, TW), jnp.float32),
            pltpu.VMEM((SLOTS * 3,), jnp.float32),
            pltpu.VMEM((SLOTS // 128, 128, AW), jnp.float32),
            pltpu.SemaphoreType.DMA,
        ],
    )(table, idx, initf, vp4)


def kernel(V_predict, L_last, V_w, V_mass_no_inf, C_shape, C_init_shape,
           V_compliance):
    f32 = jnp.float32
    vp = V_predict.astype(f32)
    table = jnp.concatenate(
        [vp, V_mass_no_inf.astype(f32), V_w.astype(f32),
         V_compliance.astype(f32), jnp.zeros((NUM_V, TW - 6), f32)], axis=1)
    idx = C_shape.astype(jnp.int32).reshape(NG, SLOTS // 128, 128)
    initf = C_init_shape.astype(f32).reshape(NG, SLOTS * 3)
    vp4 = jnp.concatenate([vp, jnp.zeros((NUM_V, AW - 3), f32)], axis=1)
    out4 = _sc_call(table, idx, initf, vp4)
    return out4[:, :3].astype(V_predict.dtype), L_last
